# baseline (speedup 1.0000x reference)
import functools

import jax
import jax.numpy as jnp
from jax import lax
from jax.experimental import pallas as pl
from jax.experimental.pallas import tpu as pltpu
from jax.experimental.pallas import tpu_sc as plsc

D_MODEL = 1024
D_FF = 2048
E = 8
TOKENS = 2048
TOP_K = 2

BLK = 256
NBLK = (TOKENS * TOP_K) // BLK + E
P = NBLK * BLK

NC = 2
NS = 16
NW = NC * NS
TPW = TOKENS // NW
CH = 32


def _router_body(x_ref, wr_ref, br_ref,
                 p0_ref, p1_ref, g0_ref, g1_ref, blk_ref):
    x = x_ref[...]
    logits = jnp.dot(x, wr_ref[...].T,
                     preferred_element_type=jnp.float32) + br_ref[...]
    lmax = jnp.max(logits, axis=-1, keepdims=True)
    ep = jnp.exp(logits - lmax)
    probs = ep / jnp.sum(ep, axis=-1, keepdims=True)

    eidx = lax.broadcasted_iota(jnp.int32, probs.shape, 1)
    big = jnp.int32(E + 1)
    m1 = jnp.max(probs, axis=-1, keepdims=True)
    i1 = jnp.min(jnp.where(probs == m1, eidx, big), axis=-1, keepdims=True)
    oh1 = eidx == i1
    masked = jnp.where(oh1, -jnp.inf, probs)
    m2 = jnp.max(masked, axis=-1, keepdims=True)
    i2 = jnp.min(jnp.where((masked == m2) & (~oh1), eidx, big),
                 axis=-1, keepdims=True)
    oh2 = eidx == i2

    den = m1 + m2
    g0_ref[...] = jnp.broadcast_to(m1 / den, (TOKENS, 16))
    g1_ref[...] = jnp.broadcast_to(m2 / den, (TOKENS, 16))

    mask = (oh1 | oh2).astype(jnp.float32)
    ri = lax.broadcasted_iota(jnp.int32, (BLK, BLK), 0)
    ci = lax.broadcasted_iota(jnp.int32, (BLK, BLK), 1)
    ltri = (ri >= ci).astype(jnp.float32)
    tot = jnp.zeros((1, E), jnp.float32)
    chunks = []
    for c in range(TOKENS // BLK):
        mc = mask[c * BLK:(c + 1) * BLK, :]
        rc = jnp.dot(ltri, mc, preferred_element_type=jnp.float32) + tot
        chunks.append(rc)
        tot = rc[BLK - 1:BLK, :]
    r = jnp.concatenate(chunks, axis=0)
    counts = tot

    nblocks = jnp.floor((counts + (BLK - 1)) / BLK)
    ui = lax.broadcasted_iota(jnp.int32, (E, E), 0)
    uj = lax.broadcasted_iota(jnp.int32, (E, E), 1)
    ustrict = (ui < uj).astype(jnp.float32)
    offb = jnp.dot(nblocks, ustrict,
                   preferred_element_type=jnp.float32)
    ends = offb + nblocks

    slot = offb * BLK + r - 1.0
    p0 = jnp.sum(jnp.where(oh1, slot, 0.0), axis=-1, keepdims=True)
    p1 = jnp.sum(jnp.where(oh2, slot, 0.0), axis=-1, keepdims=True)
    p0_ref[...] = p0.astype(jnp.int32)
    p1_ref[...] = p1.astype(jnp.int32)

    biota = lax.broadcasted_iota(jnp.int32, (NBLK, E), 0).astype(jnp.float32)
    ind = (jnp.broadcast_to(ends, (NBLK, E)) <= biota).astype(jnp.int32)
    blk_ref[...] = jnp.minimum(jnp.sum(ind, axis=-1, keepdims=True), E - 1)


def _router(x, Wr, br):
    return pl.pallas_call(
        _router_body,
        in_specs=[
            pl.BlockSpec((TOKENS, D_MODEL), lambda: (0, 0)),
            pl.BlockSpec((E, D_MODEL), lambda: (0, 0)),
            pl.BlockSpec((E,), lambda: (0,)),
        ],
        out_specs=[
            pl.BlockSpec((TOKENS, 1), lambda: (0, 0)),
            pl.BlockSpec((TOKENS, 1), lambda: (0, 0)),
            pl.BlockSpec((TOKENS, 16), lambda: (0, 0)),
            pl.BlockSpec((TOKENS, 16), lambda: (0, 0)),
            pl.BlockSpec((NBLK, 1), lambda: (0, 0)),
        ],
        out_shape=[
            jax.ShapeDtypeStruct((TOKENS, 1), jnp.int32),
            jax.ShapeDtypeStruct((TOKENS, 1), jnp.int32),
            jax.ShapeDtypeStruct((TOKENS, 16), jnp.float32),
            jax.ShapeDtypeStruct((TOKENS, 16), jnp.float32),
            jax.ShapeDtypeStruct((NBLK, 1), jnp.int32),
        ],
    )(x, Wr, br)


@functools.cache
def _scatter_kernel():
    mesh = plsc.VectorSubcoreMesh(core_axis_name="c", subcore_axis_name="s")

    @functools.partial(
        pl.kernel, mesh=mesh,
        out_type=jax.ShapeDtypeStruct((P, D_MODEL), jnp.float32),
        scratch_types=[
            pltpu.VMEM((TPW,), jnp.int32),
            pltpu.VMEM((TPW,), jnp.int32),
            pltpu.VMEM((TPW, D_MODEL), jnp.float32),
            pltpu.SemaphoreType.DMA,
        ],
    )
    def k(x_hbm, p0_hbm, p1_hbm, xg_hbm, i0_v, i1_v, xv, sem):
        wid = lax.axis_index("s") * NC + lax.axis_index("c")
        base = wid * TPW
        pltpu.sync_copy(x_hbm.at[pl.ds(base, TPW)], xv)
        pltpu.sync_copy(p0_hbm.at[pl.ds(base, TPW)], i0_v)
        pltpu.sync_copy(p1_hbm.at[pl.ds(base, TPW)], i1_v)
        c0 = pltpu.async_copy(xv, xg_hbm.at[i0_v], sem)
        c1 = pltpu.async_copy(xv, xg_hbm.at[i1_v], sem)
        c0.wait()
        c1.wait()

    return k


NF = 4
FB = D_FF // NF


def _ffn_body(blk_ref, xg_ref, w1_ref, b1_ref, w2_ref, b2_ref, out_ref):
    f = pl.program_id(1)
    h = jnp.maximum(
        jnp.dot(xg_ref[...], w1_ref[0],
                preferred_element_type=jnp.float32) + b1_ref[0], 0.0)
    y = jnp.dot(h, w2_ref[0], preferred_element_type=jnp.float32)

    @pl.when(f == 0)
    def _init():
        out_ref[...] = y + b2_ref[0]

    @pl.when(f != 0)
    def _acc():
        out_ref[...] += y


def _ffn(blk_e, xg, W1, b1, W2, b2):
    grid_spec = pltpu.PrefetchScalarGridSpec(
        num_scalar_prefetch=1,
        grid=(NBLK, NF),
        in_specs=[
            pl.BlockSpec((BLK, D_MODEL), lambda b, f, blk: (b, 0)),
            pl.BlockSpec((1, D_MODEL, FB), lambda b, f, blk: (blk[b], 0, f)),
            pl.BlockSpec((1, 1, FB), lambda b, f, blk: (blk[b], 0, f)),
            pl.BlockSpec((1, FB, D_MODEL), lambda b, f, blk: (blk[b], f, 0)),
            pl.BlockSpec((1, 1, D_MODEL), lambda b, f, blk: (blk[b], 0, 0)),
        ],
        out_specs=pl.BlockSpec((BLK, D_MODEL), lambda b, f, blk: (b, 0)),
    )
    return pl.pallas_call(
        _ffn_body,
        grid_spec=grid_spec,
        out_shape=jax.ShapeDtypeStruct((P, D_MODEL), jnp.float32),
    )(blk_e, xg, W1, b1.reshape(E, 1, D_FF), W2, b2.reshape(E, 1, D_MODEL))


@functools.cache
def _combine_kernel():
    mesh = plsc.VectorSubcoreMesh(core_axis_name="c", subcore_axis_name="s")

    @functools.partial(
        pl.kernel, mesh=mesh,
        out_type=jax.ShapeDtypeStruct((TOKENS, D_MODEL), jnp.float32),
        scratch_types=[
            pltpu.VMEM((CH,), jnp.int32),
            pltpu.VMEM((CH,), jnp.int32),
            pltpu.VMEM((CH, 16), jnp.float32),
            pltpu.VMEM((CH, 16), jnp.float32),
            pltpu.VMEM((CH, D_MODEL), jnp.float32),
            pltpu.VMEM((CH, D_MODEL), jnp.float32),
            pltpu.VMEM((CH, D_MODEL), jnp.float32),
            pltpu.SemaphoreType.DMA,
        ],
    )
    def k(y_hbm, p0_hbm, p1_hbm, g0_hbm, g1_hbm, out_hbm,
          i0_v, i1_v, g0_v, g1_v, y0_v, y1_v, o_v, sem):
        wid = lax.axis_index("s") * NC + lax.axis_index("c")
        for sc in range(TPW // CH):
            base = wid * TPW + sc * CH
            pltpu.sync_copy(p0_hbm.at[pl.ds(base, CH)], i0_v)
            pltpu.sync_copy(p1_hbm.at[pl.ds(base, CH)], i1_v)
            pltpu.sync_copy(g0_hbm.at[pl.ds(base, CH)], g0_v)
            pltpu.sync_copy(g1_hbm.at[pl.ds(base, CH)], g1_v)
            c0 = pltpu.async_copy(y_hbm.at[i0_v], y0_v, sem)
            c1 = pltpu.async_copy(y_hbm.at[i1_v], y1_v, sem)
            c0.wait()
            c1.wait()

            def body(j, carry):
                gs0 = g0_v[j, :]
                gs1 = g1_v[j, :]
                for c in range(D_MODEL // 16):
                    sl = pl.ds(c * 16, 16)
                    o_v[j, sl] = gs0 * y0_v[j, sl] + gs1 * y1_v[j, sl]
                return carry

            lax.fori_loop(0, CH, body, 0)
            pltpu.sync_copy(o_v, out_hbm.at[pl.ds(base, CH)])

    return k


@jax.jit
def kernel(x, Wr, br, W1, b1, W2, b2):
    p0, p1, g0, g1, blk2 = _router(x, Wr, br)
    p0 = p0.reshape(TOKENS)
    p1 = p1.reshape(TOKENS)
    blk_e = blk2.reshape(NBLK)
    xg = _scatter_kernel()(x, p0, p1)
    ybuf = _ffn(blk_e, xg, W1, b1, W2, b2)
    return _combine_kernel()(ybuf, p0, p1, g0, g1)


# speedup vs baseline: 1.4283x; 1.4283x over previous
import functools

import jax
import jax.numpy as jnp
from jax import lax
from jax.experimental import pallas as pl
from jax.experimental.pallas import tpu as pltpu
from jax.experimental.pallas import tpu_sc as plsc

D_MODEL = 1024
D_FF = 2048
E = 8
TOKENS = 2048
TOP_K = 2

BLK = 128
NBLK = (TOKENS * TOP_K) // BLK + E
P = NBLK * BLK

NC = 2
NS = 16
NW = NC * NS
TPW = TOKENS // NW
CH = 32


def _router_body(x_ref, wr_ref, br_ref,
                 p0_ref, p1_ref, g0_ref, g1_ref, blk_ref):
    x = x_ref[...]
    logits = jnp.dot(x, wr_ref[...].T,
                     preferred_element_type=jnp.float32) + br_ref[...]
    lmax = jnp.max(logits, axis=-1, keepdims=True)
    ep = jnp.exp(logits - lmax)
    probs = ep / jnp.sum(ep, axis=-1, keepdims=True)

    eidx = lax.broadcasted_iota(jnp.int32, probs.shape, 1)
    big = jnp.int32(E + 1)
    m1 = jnp.max(probs, axis=-1, keepdims=True)
    i1 = jnp.min(jnp.where(probs == m1, eidx, big), axis=-1, keepdims=True)
    oh1 = eidx == i1
    masked = jnp.where(oh1, -jnp.inf, probs)
    m2 = jnp.max(masked, axis=-1, keepdims=True)
    i2 = jnp.min(jnp.where((masked == m2) & (~oh1), eidx, big),
                 axis=-1, keepdims=True)
    oh2 = eidx == i2

    den = m1 + m2
    g0_ref[...] = jnp.broadcast_to(m1 / den, (TOKENS, 16))
    g1_ref[...] = jnp.broadcast_to(m2 / den, (TOKENS, 16))

    mask = (oh1 | oh2).astype(jnp.float32)
    ri = lax.broadcasted_iota(jnp.int32, (BLK, BLK), 0)
    ci = lax.broadcasted_iota(jnp.int32, (BLK, BLK), 1)
    ltri = (ri >= ci).astype(jnp.float32)
    tot = jnp.zeros((1, E), jnp.float32)
    chunks = []
    for c in range(TOKENS // BLK):
        mc = mask[c * BLK:(c + 1) * BLK, :]
        rc = jnp.dot(ltri, mc, preferred_element_type=jnp.float32) + tot
        chunks.append(rc)
        tot = rc[BLK - 1:BLK, :]
    r = jnp.concatenate(chunks, axis=0)
    counts = tot

    nblocks = jnp.floor((counts + (BLK - 1)) / BLK)
    ui = lax.broadcasted_iota(jnp.int32, (E, E), 0)
    uj = lax.broadcasted_iota(jnp.int32, (E, E), 1)
    ustrict = (ui < uj).astype(jnp.float32)
    offb = jnp.dot(nblocks, ustrict,
                   preferred_element_type=jnp.float32)
    ends = offb + nblocks

    slot = offb * BLK + r - 1.0
    p0 = jnp.sum(jnp.where(oh1, slot, 0.0), axis=-1, keepdims=True)
    p1 = jnp.sum(jnp.where(oh2, slot, 0.0), axis=-1, keepdims=True)
    p0_ref[...] = p0.astype(jnp.int32)
    p1_ref[...] = p1.astype(jnp.int32)

    biota = lax.broadcasted_iota(jnp.int32, (NBLK, E), 0).astype(jnp.float32)
    ind = (jnp.broadcast_to(ends, (NBLK, E)) <= biota).astype(jnp.int32)
    blk_ref[...] = jnp.minimum(jnp.sum(ind, axis=-1, keepdims=True), E - 1)


def _router(x, Wr, br):
    return pl.pallas_call(
        _router_body,
        in_specs=[
            pl.BlockSpec((TOKENS, D_MODEL), lambda: (0, 0)),
            pl.BlockSpec((E, D_MODEL), lambda: (0, 0)),
            pl.BlockSpec((E,), lambda: (0,)),
        ],
        out_specs=[
            pl.BlockSpec((TOKENS, 1), lambda: (0, 0)),
            pl.BlockSpec((TOKENS, 1), lambda: (0, 0)),
            pl.BlockSpec((TOKENS, 16), lambda: (0, 0)),
            pl.BlockSpec((TOKENS, 16), lambda: (0, 0)),
            pl.BlockSpec((NBLK, 1), lambda: (0, 0)),
        ],
        out_shape=[
            jax.ShapeDtypeStruct((TOKENS, 1), jnp.int32),
            jax.ShapeDtypeStruct((TOKENS, 1), jnp.int32),
            jax.ShapeDtypeStruct((TOKENS, 16), jnp.float32),
            jax.ShapeDtypeStruct((TOKENS, 16), jnp.float32),
            jax.ShapeDtypeStruct((NBLK, 1), jnp.int32),
        ],
    )(x, Wr, br)


@functools.cache
def _scatter_kernel():
    mesh = plsc.VectorSubcoreMesh(core_axis_name="c", subcore_axis_name="s")

    @functools.partial(
        pl.kernel, mesh=mesh,
        out_type=jax.ShapeDtypeStruct((P, D_MODEL), jnp.float32),
        scratch_types=[
            pltpu.VMEM((TPW,), jnp.int32),
            pltpu.VMEM((TPW,), jnp.int32),
            pltpu.VMEM((TPW, D_MODEL), jnp.float32),
            pltpu.SemaphoreType.DMA,
        ],
    )
    def k(x_hbm, p0_hbm, p1_hbm, xg_hbm, i0_v, i1_v, xv, sem):
        wid = lax.axis_index("s") * NC + lax.axis_index("c")
        base = wid * TPW
        pltpu.sync_copy(x_hbm.at[pl.ds(base, TPW)], xv)
        pltpu.sync_copy(p0_hbm.at[pl.ds(base, TPW)], i0_v)
        pltpu.sync_copy(p1_hbm.at[pl.ds(base, TPW)], i1_v)
        c0 = pltpu.async_copy(xv, xg_hbm.at[i0_v], sem)
        c1 = pltpu.async_copy(xv, xg_hbm.at[i1_v], sem)
        c0.wait()
        c1.wait()

    return k


def _ffn_body(blk_ref, xg_ref, w1_ref, b1_ref, w2_ref, b2_ref, out_ref):
    h = jnp.maximum(
        jnp.dot(xg_ref[...], w1_ref[0],
                preferred_element_type=jnp.float32) + b1_ref[0], 0.0)
    out_ref[...] = (jnp.dot(h, w2_ref[0], preferred_element_type=jnp.float32)
                    + b2_ref[0])


def _ffn(blk_e, xg, W1, b1, W2, b2):
    grid_spec = pltpu.PrefetchScalarGridSpec(
        num_scalar_prefetch=1,
        grid=(NBLK,),
        in_specs=[
            pl.BlockSpec((BLK, D_MODEL), lambda b, blk: (b, 0)),
            pl.BlockSpec((1, D_MODEL, D_FF), lambda b, blk: (blk[b], 0, 0)),
            pl.BlockSpec((1, 1, D_FF), lambda b, blk: (blk[b], 0, 0)),
            pl.BlockSpec((1, D_FF, D_MODEL), lambda b, blk: (blk[b], 0, 0)),
            pl.BlockSpec((1, 1, D_MODEL), lambda b, blk: (blk[b], 0, 0)),
        ],
        out_specs=pl.BlockSpec((BLK, D_MODEL), lambda b, blk: (b, 0)),
    )
    return pl.pallas_call(
        _ffn_body,
        grid_spec=grid_spec,
        out_shape=jax.ShapeDtypeStruct((P, D_MODEL), jnp.float32),
    )(blk_e, xg, W1, b1.reshape(E, 1, D_FF), W2, b2.reshape(E, 1, D_MODEL))


@functools.cache
def _combine_kernel():
    mesh = plsc.VectorSubcoreMesh(core_axis_name="c", subcore_axis_name="s")

    @functools.partial(
        pl.kernel, mesh=mesh,
        out_type=jax.ShapeDtypeStruct((TOKENS, D_MODEL), jnp.float32),
        scratch_types=[
            pltpu.VMEM((CH,), jnp.int32),
            pltpu.VMEM((CH,), jnp.int32),
            pltpu.VMEM((CH, 16), jnp.float32),
            pltpu.VMEM((CH, 16), jnp.float32),
            pltpu.VMEM((CH, D_MODEL), jnp.float32),
            pltpu.VMEM((CH, D_MODEL), jnp.float32),
            pltpu.VMEM((CH, D_MODEL), jnp.float32),
            pltpu.SemaphoreType.DMA,
        ],
    )
    def k(y_hbm, p0_hbm, p1_hbm, g0_hbm, g1_hbm, out_hbm,
          i0_v, i1_v, g0_v, g1_v, y0_v, y1_v, o_v, sem):
        wid = lax.axis_index("s") * NC + lax.axis_index("c")
        for sc in range(TPW // CH):
            base = wid * TPW + sc * CH
            pltpu.sync_copy(p0_hbm.at[pl.ds(base, CH)], i0_v)
            pltpu.sync_copy(p1_hbm.at[pl.ds(base, CH)], i1_v)
            pltpu.sync_copy(g0_hbm.at[pl.ds(base, CH)], g0_v)
            pltpu.sync_copy(g1_hbm.at[pl.ds(base, CH)], g1_v)
            c0 = pltpu.async_copy(y_hbm.at[i0_v], y0_v, sem)
            c1 = pltpu.async_copy(y_hbm.at[i1_v], y1_v, sem)
            c0.wait()
            c1.wait()

            def body(j, carry):
                gs0 = g0_v[j, :]
                gs1 = g1_v[j, :]
                for c in range(D_MODEL // 16):
                    sl = pl.ds(c * 16, 16)
                    o_v[j, sl] = gs0 * y0_v[j, sl] + gs1 * y1_v[j, sl]
                return carry

            lax.fori_loop(0, CH, body, 0)
            pltpu.sync_copy(o_v, out_hbm.at[pl.ds(base, CH)])

    return k


@jax.jit
def kernel(x, Wr, br, W1, b1, W2, b2):
    p0, p1, g0, g1, blk2 = _router(x, Wr, br)
    p0 = p0.reshape(TOKENS)
    p1 = p1.reshape(TOKENS)
    blk_e = blk2.reshape(NBLK)
    xg = _scatter_kernel()(x, p0, p1)
    ybuf = _ffn(blk_e, xg, W1, b1, W2, b2)
    return _combine_kernel()(ybuf, p0, p1, g0, g1)


# speedup vs baseline: 1.5260x; 1.0684x over previous
import functools

import jax
import jax.numpy as jnp
from jax import lax
from jax.experimental import pallas as pl
from jax.experimental.pallas import tpu as pltpu
from jax.experimental.pallas import tpu_sc as plsc

D_MODEL = 1024
D_FF = 2048
E = 8
TOKENS = 2048
TOP_K = 2

BLK = 256
NBLK = (TOKENS * TOP_K) // BLK + E
P = NBLK * BLK

NC = 2
NS = 16
NW = NC * NS
TPW = TOKENS // NW
CH = 32


def _router_body(x_ref, wr_ref, br_ref,
                 p0_ref, p1_ref, g0_ref, g1_ref, blk_ref):
    x = x_ref[...]
    logits = jnp.dot(x, wr_ref[...].T,
                     preferred_element_type=jnp.float32) + br_ref[...]
    lmax = jnp.max(logits, axis=-1, keepdims=True)
    ep = jnp.exp(logits - lmax)
    probs = ep / jnp.sum(ep, axis=-1, keepdims=True)

    eidx = lax.broadcasted_iota(jnp.int32, probs.shape, 1)
    big = jnp.int32(E + 1)
    m1 = jnp.max(probs, axis=-1, keepdims=True)
    i1 = jnp.min(jnp.where(probs == m1, eidx, big), axis=-1, keepdims=True)
    oh1 = eidx == i1
    masked = jnp.where(oh1, -jnp.inf, probs)
    m2 = jnp.max(masked, axis=-1, keepdims=True)
    i2 = jnp.min(jnp.where((masked == m2) & (~oh1), eidx, big),
                 axis=-1, keepdims=True)
    oh2 = eidx == i2

    den = m1 + m2
    g0_ref[...] = jnp.broadcast_to(m1 / den, (TOKENS, 16))
    g1_ref[...] = jnp.broadcast_to(m2 / den, (TOKENS, 16))

    mask = (oh1 | oh2).astype(jnp.float32)
    ri = lax.broadcasted_iota(jnp.int32, (BLK, BLK), 0)
    ci = lax.broadcasted_iota(jnp.int32, (BLK, BLK), 1)
    ltri = (ri >= ci).astype(jnp.float32)
    tot = jnp.zeros((1, E), jnp.float32)
    chunks = []
    for c in range(TOKENS // BLK):
        mc = mask[c * BLK:(c + 1) * BLK, :]
        rc = jnp.dot(ltri, mc, preferred_element_type=jnp.float32) + tot
        chunks.append(rc)
        tot = rc[BLK - 1:BLK, :]
    r = jnp.concatenate(chunks, axis=0)
    counts = tot

    nblocks = jnp.floor((counts + (BLK - 1)) / BLK)
    ui = lax.broadcasted_iota(jnp.int32, (E, E), 0)
    uj = lax.broadcasted_iota(jnp.int32, (E, E), 1)
    ustrict = (ui < uj).astype(jnp.float32)
    offb = jnp.dot(nblocks, ustrict,
                   preferred_element_type=jnp.float32)
    ends = offb + nblocks

    slot = offb * BLK + r - 1.0
    p0 = jnp.sum(jnp.where(oh1, slot, 0.0), axis=-1, keepdims=True)
    p1 = jnp.sum(jnp.where(oh2, slot, 0.0), axis=-1, keepdims=True)
    p0_ref[...] = p0.astype(jnp.int32)
    p1_ref[...] = p1.astype(jnp.int32)

    biota = lax.broadcasted_iota(jnp.int32, (NBLK + 1, E), 0).astype(jnp.float32)
    ind = (jnp.broadcast_to(ends, (NBLK + 1, E)) <= biota).astype(jnp.int32)
    bmap = jnp.minimum(jnp.sum(ind, axis=-1, keepdims=True), E - 1)
    used = jnp.sum(nblocks).astype(jnp.int32)
    last = lax.broadcasted_iota(jnp.int32, (NBLK + 1, 1), 0) == NBLK
    blk_ref[...] = jnp.where(last, used, bmap)


def _router(x, Wr, br):
    return pl.pallas_call(
        _router_body,
        in_specs=[
            pl.BlockSpec((TOKENS, D_MODEL), lambda: (0, 0)),
            pl.BlockSpec((E, D_MODEL), lambda: (0, 0)),
            pl.BlockSpec((E,), lambda: (0,)),
        ],
        out_specs=[
            pl.BlockSpec((TOKENS, 1), lambda: (0, 0)),
            pl.BlockSpec((TOKENS, 1), lambda: (0, 0)),
            pl.BlockSpec((TOKENS, 16), lambda: (0, 0)),
            pl.BlockSpec((TOKENS, 16), lambda: (0, 0)),
            pl.BlockSpec((NBLK + 1, 1), lambda: (0, 0)),
        ],
        out_shape=[
            jax.ShapeDtypeStruct((TOKENS, 1), jnp.int32),
            jax.ShapeDtypeStruct((TOKENS, 1), jnp.int32),
            jax.ShapeDtypeStruct((TOKENS, 16), jnp.float32),
            jax.ShapeDtypeStruct((TOKENS, 16), jnp.float32),
            jax.ShapeDtypeStruct((NBLK + 1, 1), jnp.int32),
        ],
    )(x, Wr, br)


@functools.cache
def _scatter_kernel():
    mesh = plsc.VectorSubcoreMesh(core_axis_name="c", subcore_axis_name="s")

    @functools.partial(
        pl.kernel, mesh=mesh,
        out_type=jax.ShapeDtypeStruct((P, D_MODEL), jnp.float32),
        scratch_types=[
            pltpu.VMEM((TPW,), jnp.int32),
            pltpu.VMEM((TPW,), jnp.int32),
            pltpu.VMEM((TPW, D_MODEL), jnp.float32),
            pltpu.SemaphoreType.DMA,
        ],
    )
    def k(x_hbm, p0_hbm, p1_hbm, xg_hbm, i0_v, i1_v, xv, sem):
        wid = lax.axis_index("s") * NC + lax.axis_index("c")
        base = wid * TPW
        pltpu.sync_copy(x_hbm.at[pl.ds(base, TPW)], xv)
        pltpu.sync_copy(p0_hbm.at[pl.ds(base, TPW)], i0_v)
        pltpu.sync_copy(p1_hbm.at[pl.ds(base, TPW)], i1_v)
        c0 = pltpu.async_copy(xv, xg_hbm.at[i0_v], sem)
        c1 = pltpu.async_copy(xv, xg_hbm.at[i1_v], sem)
        c0.wait()
        c1.wait()

    return k


def _ffn_body(blk_ref, xg_ref, w1_ref, b1_ref, w2_ref, b2_ref, out_ref):
    @pl.when(pl.program_id(0) < blk_ref[NBLK])
    def _compute():
        h = jnp.maximum(
            jnp.dot(xg_ref[...], w1_ref[0],
                    preferred_element_type=jnp.float32) + b1_ref[0], 0.0)
        out_ref[...] = (jnp.dot(h, w2_ref[0],
                                preferred_element_type=jnp.float32)
                        + b2_ref[0])


def _ffn(blk_e, xg, W1, b1, W2, b2):
    grid_spec = pltpu.PrefetchScalarGridSpec(
        num_scalar_prefetch=1,
        grid=(NBLK,),
        in_specs=[
            pl.BlockSpec((BLK, D_MODEL), lambda b, blk: (b, 0)),
            pl.BlockSpec((1, D_MODEL, D_FF), lambda b, blk: (blk[b], 0, 0)),
            pl.BlockSpec((1, 1, D_FF), lambda b, blk: (blk[b], 0, 0)),
            pl.BlockSpec((1, D_FF, D_MODEL), lambda b, blk: (blk[b], 0, 0)),
            pl.BlockSpec((1, 1, D_MODEL), lambda b, blk: (blk[b], 0, 0)),
        ],
        out_specs=pl.BlockSpec((BLK, D_MODEL), lambda b, blk: (b, 0)),
    )
    return pl.pallas_call(
        _ffn_body,
        grid_spec=grid_spec,
        out_shape=jax.ShapeDtypeStruct((P, D_MODEL), jnp.float32),
    )(blk_e, xg, W1, b1.reshape(E, 1, D_FF), W2, b2.reshape(E, 1, D_MODEL))


@functools.cache
def _combine_kernel():
    mesh = plsc.VectorSubcoreMesh(core_axis_name="c", subcore_axis_name="s")

    @functools.partial(
        pl.kernel, mesh=mesh,
        out_type=jax.ShapeDtypeStruct((TOKENS, D_MODEL), jnp.float32),
        scratch_types=[
            pltpu.VMEM((CH,), jnp.int32),
            pltpu.VMEM((CH,), jnp.int32),
            pltpu.VMEM((CH, 16), jnp.float32),
            pltpu.VMEM((CH, 16), jnp.float32),
            pltpu.VMEM((CH, D_MODEL), jnp.float32),
            pltpu.VMEM((CH, D_MODEL), jnp.float32),
            pltpu.VMEM((CH, D_MODEL), jnp.float32),
            pltpu.SemaphoreType.DMA,
        ],
    )
    def k(y_hbm, p0_hbm, p1_hbm, g0_hbm, g1_hbm, out_hbm,
          i0_v, i1_v, g0_v, g1_v, y0_v, y1_v, o_v, sem):
        wid = lax.axis_index("s") * NC + lax.axis_index("c")
        for sc in range(TPW // CH):
            base = wid * TPW + sc * CH
            pltpu.sync_copy(p0_hbm.at[pl.ds(base, CH)], i0_v)
            pltpu.sync_copy(p1_hbm.at[pl.ds(base, CH)], i1_v)
            pltpu.sync_copy(g0_hbm.at[pl.ds(base, CH)], g0_v)
            pltpu.sync_copy(g1_hbm.at[pl.ds(base, CH)], g1_v)
            c0 = pltpu.async_copy(y_hbm.at[i0_v], y0_v, sem)
            c1 = pltpu.async_copy(y_hbm.at[i1_v], y1_v, sem)
            c0.wait()
            c1.wait()

            def body(j, carry):
                gs0 = g0_v[j, :]
                gs1 = g1_v[j, :]
                for c in range(D_MODEL // 16):
                    sl = pl.ds(c * 16, 16)
                    o_v[j, sl] = gs0 * y0_v[j, sl] + gs1 * y1_v[j, sl]
                return carry

            lax.fori_loop(0, CH, body, 0)
            pltpu.sync_copy(o_v, out_hbm.at[pl.ds(base, CH)])

    return k


@jax.jit
def kernel(x, Wr, br, W1, b1, W2, b2):
    p0, p1, g0, g1, blk2 = _router(x, Wr, br)
    p0 = p0.reshape(TOKENS)
    p1 = p1.reshape(TOKENS)
    blk_e = blk2.reshape(NBLK + 1)
    xg = _scatter_kernel()(x, p0, p1)
    ybuf = _ffn(blk_e, xg, W1, b1, W2, b2)
    return _combine_kernel()(ybuf, p0, p1, g0, g1)


# speedup vs baseline: 1.5744x; 1.0317x over previous
import functools

import jax
import jax.numpy as jnp
from jax import lax
from jax.experimental import pallas as pl
from jax.experimental.pallas import tpu as pltpu
from jax.experimental.pallas import tpu_sc as plsc

D_MODEL = 1024
D_FF = 2048
E = 8
TOKENS = 2048
TOP_K = 2

BLK = 256
NBLK = (TOKENS * TOP_K) // BLK + E
P = NBLK * BLK

NC = 2
NS = 16
NW = NC * NS
TPW = TOKENS // NW
CH = 16


def _router_body(x_ref, wr_ref, br_ref,
                 p0_ref, p1_ref, g0_ref, g1_ref, blk_ref):
    x = x_ref[...]
    logits = jnp.dot(x, wr_ref[...].T,
                     preferred_element_type=jnp.float32) + br_ref[...]
    lmax = jnp.max(logits, axis=-1, keepdims=True)
    ep = jnp.exp(logits - lmax)
    probs = ep / jnp.sum(ep, axis=-1, keepdims=True)

    eidx = lax.broadcasted_iota(jnp.int32, probs.shape, 1)
    big = jnp.int32(E + 1)
    m1 = jnp.max(probs, axis=-1, keepdims=True)
    i1 = jnp.min(jnp.where(probs == m1, eidx, big), axis=-1, keepdims=True)
    oh1 = eidx == i1
    masked = jnp.where(oh1, -jnp.inf, probs)
    m2 = jnp.max(masked, axis=-1, keepdims=True)
    i2 = jnp.min(jnp.where((masked == m2) & (~oh1), eidx, big),
                 axis=-1, keepdims=True)
    oh2 = eidx == i2

    den = m1 + m2
    g0_ref[...] = jnp.broadcast_to(m1 / den, (TOKENS, 16))
    g1_ref[...] = jnp.broadcast_to(m2 / den, (TOKENS, 16))

    mask = (oh1 | oh2).astype(jnp.float32)
    ri = lax.broadcasted_iota(jnp.int32, (BLK, BLK), 0)
    ci = lax.broadcasted_iota(jnp.int32, (BLK, BLK), 1)
    ltri = (ri >= ci).astype(jnp.float32)
    tot = jnp.zeros((1, E), jnp.float32)
    chunks = []
    for c in range(TOKENS // BLK):
        mc = mask[c * BLK:(c + 1) * BLK, :]
        rc = jnp.dot(ltri, mc, preferred_element_type=jnp.float32) + tot
        chunks.append(rc)
        tot = rc[BLK - 1:BLK, :]
    r = jnp.concatenate(chunks, axis=0)
    counts = tot

    nblocks = jnp.floor((counts + (BLK - 1)) / BLK)
    ui = lax.broadcasted_iota(jnp.int32, (E, E), 0)
    uj = lax.broadcasted_iota(jnp.int32, (E, E), 1)
    ustrict = (ui < uj).astype(jnp.float32)
    offb = jnp.dot(nblocks, ustrict,
                   preferred_element_type=jnp.float32)
    ends = offb + nblocks

    slot = offb * BLK + r - 1.0
    p0 = jnp.sum(jnp.where(oh1, slot, 0.0), axis=-1, keepdims=True)
    p1 = jnp.sum(jnp.where(oh2, slot, 0.0), axis=-1, keepdims=True)
    p0_ref[...] = p0.astype(jnp.int32)
    p1_ref[...] = p1.astype(jnp.int32)

    biota = lax.broadcasted_iota(jnp.int32, (NBLK + 1, E), 0).astype(jnp.float32)
    ind = (jnp.broadcast_to(ends, (NBLK + 1, E)) <= biota).astype(jnp.int32)
    bmap = jnp.minimum(jnp.sum(ind, axis=-1, keepdims=True), E - 1)
    used = jnp.sum(nblocks).astype(jnp.int32)
    last = lax.broadcasted_iota(jnp.int32, (NBLK + 1, 1), 0) == NBLK
    blk_ref[...] = jnp.where(last, used, bmap)


def _router(x, Wr, br):
    return pl.pallas_call(
        _router_body,
        in_specs=[
            pl.BlockSpec((TOKENS, D_MODEL), lambda: (0, 0)),
            pl.BlockSpec((E, D_MODEL), lambda: (0, 0)),
            pl.BlockSpec((E,), lambda: (0,)),
        ],
        out_specs=[
            pl.BlockSpec((TOKENS, 1), lambda: (0, 0)),
            pl.BlockSpec((TOKENS, 1), lambda: (0, 0)),
            pl.BlockSpec((TOKENS, 16), lambda: (0, 0)),
            pl.BlockSpec((TOKENS, 16), lambda: (0, 0)),
            pl.BlockSpec((NBLK + 1, 1), lambda: (0, 0)),
        ],
        out_shape=[
            jax.ShapeDtypeStruct((TOKENS, 1), jnp.int32),
            jax.ShapeDtypeStruct((TOKENS, 1), jnp.int32),
            jax.ShapeDtypeStruct((TOKENS, 16), jnp.float32),
            jax.ShapeDtypeStruct((TOKENS, 16), jnp.float32),
            jax.ShapeDtypeStruct((NBLK + 1, 1), jnp.int32),
        ],
    )(x, Wr, br)


SCH = TPW // 2


@functools.cache
def _scatter_kernel():
    mesh = plsc.VectorSubcoreMesh(core_axis_name="c", subcore_axis_name="s")

    @functools.partial(
        pl.kernel, mesh=mesh,
        out_type=jax.ShapeDtypeStruct((P, D_MODEL), jnp.float32),
        scratch_types=[
            pltpu.VMEM((TPW,), jnp.int32),
            pltpu.VMEM((TPW,), jnp.int32),
            pltpu.VMEM((SCH, D_MODEL), jnp.float32),
            pltpu.VMEM((SCH, D_MODEL), jnp.float32),
            pltpu.SemaphoreType.DMA,
            pltpu.SemaphoreType.DMA,
            pltpu.SemaphoreType.DMA,
        ],
    )
    def k(x_hbm, p0_hbm, p1_hbm, xg_hbm, i0_v, i1_v, xa, xb, sa, sb, ss):
        wid = lax.axis_index("s") * NC + lax.axis_index("c")
        base = wid * TPW
        pltpu.sync_copy(p0_hbm.at[pl.ds(base, TPW)], i0_v)
        pltpu.sync_copy(p1_hbm.at[pl.ds(base, TPW)], i1_v)
        la = pltpu.async_copy(x_hbm.at[pl.ds(base, SCH)], xa, sa)
        lb = pltpu.async_copy(x_hbm.at[pl.ds(base + SCH, SCH)], xb, sb)
        la.wait()
        w0 = pltpu.async_copy(xa, xg_hbm.at[i0_v.at[pl.ds(0, SCH)]], ss)
        w1 = pltpu.async_copy(xa, xg_hbm.at[i1_v.at[pl.ds(0, SCH)]], ss)
        lb.wait()
        w2 = pltpu.async_copy(xb, xg_hbm.at[i0_v.at[pl.ds(SCH, SCH)]], ss)
        w3 = pltpu.async_copy(xb, xg_hbm.at[i1_v.at[pl.ds(SCH, SCH)]], ss)
        w0.wait()
        w1.wait()
        w2.wait()
        w3.wait()

    return k


def _ffn_body(blk_ref, xg_ref, w1_ref, b1_ref, w2_ref, b2_ref, out_ref):
    @pl.when(pl.program_id(0) < blk_ref[NBLK])
    def _compute():
        h = jnp.maximum(
            jnp.dot(xg_ref[...], w1_ref[0],
                    preferred_element_type=jnp.float32) + b1_ref[0], 0.0)
        out_ref[...] = (jnp.dot(h, w2_ref[0],
                                preferred_element_type=jnp.float32)
                        + b2_ref[0])


def _ffn(blk_e, xg, W1, b1, W2, b2):
    grid_spec = pltpu.PrefetchScalarGridSpec(
        num_scalar_prefetch=1,
        grid=(NBLK,),
        in_specs=[
            pl.BlockSpec((BLK, D_MODEL), lambda b, blk: (b, 0)),
            pl.BlockSpec((1, D_MODEL, D_FF), lambda b, blk: (blk[b], 0, 0)),
            pl.BlockSpec((1, 1, D_FF), lambda b, blk: (blk[b], 0, 0)),
            pl.BlockSpec((1, D_FF, D_MODEL), lambda b, blk: (blk[b], 0, 0)),
            pl.BlockSpec((1, 1, D_MODEL), lambda b, blk: (blk[b], 0, 0)),
        ],
        out_specs=pl.BlockSpec((BLK, D_MODEL), lambda b, blk: (b, 0)),
    )
    return pl.pallas_call(
        _ffn_body,
        grid_spec=grid_spec,
        out_shape=jax.ShapeDtypeStruct((P, D_MODEL), jnp.float32),
    )(blk_e, xg, W1, b1.reshape(E, 1, D_FF), W2, b2.reshape(E, 1, D_MODEL))


NCH = TPW // CH


@functools.cache
def _combine_kernel():
    mesh = plsc.VectorSubcoreMesh(core_axis_name="c", subcore_axis_name="s")

    @functools.partial(
        pl.kernel, mesh=mesh,
        out_type=jax.ShapeDtypeStruct((TOKENS, D_MODEL), jnp.float32),
        scratch_types=[
            pltpu.VMEM((TPW,), jnp.int32),
            pltpu.VMEM((TPW,), jnp.int32),
            pltpu.VMEM((TPW, 16), jnp.float32),
            pltpu.VMEM((TPW, 16), jnp.float32),
            pltpu.VMEM((CH, D_MODEL), jnp.float32),
            pltpu.VMEM((CH, D_MODEL), jnp.float32),
            pltpu.VMEM((CH, D_MODEL), jnp.float32),
            pltpu.VMEM((CH, D_MODEL), jnp.float32),
            pltpu.VMEM((CH, D_MODEL), jnp.float32),
            pltpu.VMEM((CH, D_MODEL), jnp.float32),
            pltpu.SemaphoreType.DMA,
            pltpu.SemaphoreType.DMA,
        ],
    )
    def k(y_hbm, p0_hbm, p1_hbm, g0_hbm, g1_hbm, out_hbm,
          i0_v, i1_v, g0_v, g1_v, y0a, y1a, y0b, y1b, oa, ob, sa, sb):
        wid = lax.axis_index("s") * NC + lax.axis_index("c")
        base = wid * TPW
        pltpu.sync_copy(p0_hbm.at[pl.ds(base, TPW)], i0_v)
        pltpu.sync_copy(p1_hbm.at[pl.ds(base, TPW)], i1_v)
        pltpu.sync_copy(g0_hbm.at[pl.ds(base, TPW)], g0_v)
        pltpu.sync_copy(g1_hbm.at[pl.ds(base, TPW)], g1_v)

        ybufs = ((y0a, y1a, sa), (y0b, y1b, sb))
        obufs = (oa, ob)

        def issue(i):
            y0s, y1s, sem = ybufs[i % 2]
            c0 = pltpu.async_copy(y_hbm.at[i0_v.at[pl.ds(i * CH, CH)]],
                                  y0s, sem)
            c1 = pltpu.async_copy(y_hbm.at[i1_v.at[pl.ds(i * CH, CH)]],
                                  y1s, sem)
            return c0, c1

        pend = [issue(0), issue(1)]
        for i in range(NCH):
            y0s, y1s, _ = ybufs[i % 2]
            o_v = obufs[i % 2]
            c0, c1 = pend[i % 2]
            c0.wait()
            c1.wait()

            def body(j, carry):
                gs0 = g0_v[i * CH + j, :]
                gs1 = g1_v[i * CH + j, :]
                for c in range(D_MODEL // 16):
                    sl = pl.ds(c * 16, 16)
                    o_v[j, sl] = gs0 * y0s[j, sl] + gs1 * y1s[j, sl]
                return carry

            lax.fori_loop(0, CH, body, 0)
            if i + 2 < NCH:
                pend[i % 2] = issue(i + 2)
            pltpu.sync_copy(o_v, out_hbm.at[pl.ds(base + i * CH, CH)])

    return k


@jax.jit
def kernel(x, Wr, br, W1, b1, W2, b2):
    p0, p1, g0, g1, blk2 = _router(x, Wr, br)
    p0 = p0.reshape(TOKENS)
    p1 = p1.reshape(TOKENS)
    blk_e = blk2.reshape(NBLK + 1)
    xg = _scatter_kernel()(x, p0, p1)
    ybuf = _ffn(blk_e, xg, W1, b1, W2, b2)
    return _combine_kernel()(ybuf, p0, p1, g0, g1)
